# baseline (speedup 1.0000x reference)
import functools

import jax
import jax.numpy as jnp
from jax import lax
from jax.experimental import pallas as pl
from jax.experimental.pallas import tpu as pltpu
from jax.experimental.pallas import tpu_sc as plsc

B = 4096
D = 64

_NC = 2
_NS = 16
_NW = _NC * _NS
_BPW = B // _NW


@functools.cache
def _make_sc_gather():
    mesh = plsc.VectorSubcoreMesh(core_axis_name="c", subcore_axis_name="s")

    @functools.partial(
        pl.kernel,
        mesh=mesh,
        out_type=jax.ShapeDtypeStruct((B, D), jnp.float32),
        scratch_types=[
            pltpu.VMEM((_BPW,), jnp.int32),
            pltpu.VMEM((_BPW, D), jnp.float32),
            pltpu.SemaphoreType.DMA,
        ],
        compiler_params=pltpu.CompilerParams(use_tc_tiling_on_sc=False),
    )
    def _sc_gather(idx_hbm, tab_hbm, out_hbm, idx_v, rows_v, sem):
        wid = lax.axis_index("s") * _NC + lax.axis_index("c")
        base = wid * _BPW
        pltpu.sync_copy(idx_hbm.at[pl.ds(base, _BPW)], idx_v)
        pltpu.async_copy(tab_hbm.at[idx_v], rows_v, sem).wait()
        pltpu.sync_copy(rows_v, out_hbm.at[pl.ds(base, _BPW)])

    return _sc_gather


_NSEM = 8


def _tc_gather_body(idx_ref, tab_ref, out_ref, vmem, sems, osem):
    def issue(k, _):
        for j in range(_NSEM):
            row = k * _NSEM + j
            pltpu.make_async_copy(
                tab_ref.at[pl.ds(idx_ref[row], 1)],
                vmem.at[pl.ds(row, 1)],
                sems.at[j],
            ).start()
        return 0
    lax.fori_loop(0, B // _NSEM, issue, 0, unroll=True)

    nrows = B // _NSEM
    for j in range(_NSEM):
        pltpu.make_async_copy(
            tab_ref.at[pl.ds(0, nrows)],
            vmem.at[pl.ds(0, nrows)],
            sems.at[j],
        ).wait()

    cp = pltpu.make_async_copy(vmem, out_ref, osem)
    cp.start()
    cp.wait()


_tc_gather = pl.pallas_call(
    _tc_gather_body,
    in_specs=[
        pl.BlockSpec(memory_space=pltpu.SMEM),
        pl.BlockSpec(memory_space=pl.ANY),
    ],
    out_specs=pl.BlockSpec(memory_space=pl.ANY),
    out_shape=jax.ShapeDtypeStruct((B, D), jnp.float32),
    scratch_shapes=[
        pltpu.VMEM((B, D), jnp.float32),
        pltpu.SemaphoreType.DMA((_NSEM,)),
        pltpu.SemaphoreType.DMA,
    ],
)


_BM = 512
_BN = 1024


def _mm_body(u_ref, i_ref, o_ref):
    o_ref[...] = lax.dot_general(
        u_ref[...], i_ref[...],
        (((1,), (1,)), ((), ())),
        preferred_element_type=jnp.float32,
    )


_matmul = pl.pallas_call(
    _mm_body,
    grid=(B // _BM, B // _BN),
    in_specs=[
        pl.BlockSpec((_BM, D), lambda i, j: (i, 0)),
        pl.BlockSpec((_BN, D), lambda i, j: (j, 0)),
    ],
    out_specs=pl.BlockSpec((_BM, _BN), lambda i, j: (i, j)),
    out_shape=jax.ShapeDtypeStruct((B, B), jnp.float32),
)


@jax.jit
def kernel(user_indices, item_indices, user_table, item_table):
    item_embs = _make_sc_gather()(item_indices.astype(jnp.int32), item_table)
    user_embs = _tc_gather(user_indices.astype(jnp.int32), user_table)
    return _matmul(user_embs, item_embs)


# speedup vs baseline: 1.3760x; 1.3760x over previous
import functools

import jax
import jax.numpy as jnp
from jax import lax
from jax.experimental import pallas as pl
from jax.experimental.pallas import tpu as pltpu

B = 4096
D = 64
_BM = 512
_BN = 1024
_NI = B // _BM
_NJ = B // _BN


def _body(uidx_ref, iidx_ref, utab_ref, itab_ref, o_ref,
          uvmem, ivmem, usems, isems):
    i = pl.program_id(0)
    j = pl.program_id(1)

    @pl.when(jnp.logical_and(i == 0, j == 0))
    def _issue():
        def u_rows(blk):
            def go(k, _):
                row = blk * _BM + k
                pltpu.make_async_copy(
                    utab_ref.at[pl.ds(uidx_ref[row], 1)],
                    uvmem.at[pl.ds(row, 1)],
                    usems.at[blk],
                ).start()
                return 0
            lax.fori_loop(0, _BM, go, 0, unroll=True)

        def i_rows(blk):
            def go(k, _):
                row = blk * _BN + k
                pltpu.make_async_copy(
                    itab_ref.at[pl.ds(iidx_ref[row], 1)],
                    ivmem.at[pl.ds(row, 1)],
                    isems.at[blk],
                ).start()
                return 0
            lax.fori_loop(0, _BN, go, 0, unroll=True)

        u_rows(0)
        for jb in range(_NJ):
            i_rows(jb)
        for ib in range(1, _NI):
            u_rows(ib)

    @pl.when(j == 0)
    def _wait_u():
        pltpu.make_async_copy(
            utab_ref.at[pl.ds(0, _BM)], uvmem.at[pl.ds(0, _BM)], usems.at[i]
        ).wait()

    @pl.when(i == 0)
    def _wait_i():
        pltpu.make_async_copy(
            itab_ref.at[pl.ds(0, _BN)], ivmem.at[pl.ds(0, _BN)], isems.at[j]
        ).wait()

    u = uvmem[pl.ds(i * _BM, _BM), :]
    v = ivmem[pl.ds(j * _BN, _BN), :]
    o_ref[...] = lax.dot_general(
        u, v, (((1,), (1,)), ((), ())), preferred_element_type=jnp.float32)


_fused = pl.pallas_call(
    _body,
    grid=(_NI, _NJ),
    in_specs=[
        pl.BlockSpec(memory_space=pltpu.SMEM),
        pl.BlockSpec(memory_space=pltpu.SMEM),
        pl.BlockSpec(memory_space=pl.ANY),
        pl.BlockSpec(memory_space=pl.ANY),
    ],
    out_specs=pl.BlockSpec((_BM, _BN), lambda i, j: (i, j)),
    out_shape=jax.ShapeDtypeStruct((B, B), jnp.float32),
    scratch_shapes=[
        pltpu.VMEM((B, D), jnp.float32),
        pltpu.VMEM((B, D), jnp.float32),
        pltpu.SemaphoreType.DMA((_NI,)),
        pltpu.SemaphoreType.DMA((_NJ,)),
    ],
)


@jax.jit
def kernel(user_indices, item_indices, user_table, item_table):
    return _fused(
        user_indices.astype(jnp.int32), item_indices.astype(jnp.int32),
        user_table, item_table)
